# baseline (speedup 1.0000x reference)
import jax
import jax.numpy as jnp
from jax import lax
from jax.experimental import pallas as pl
from jax.experimental.pallas import tpu as pltpu
from jax.experimental.pallas import tpu_sc as plsc

_E = 64
_D = 2048
_ALPHA = 0.01
_T = 2048
_C = 512
_NW = 32
_N = 16384
_RPW = _N // _NW


def _tc_chunk(x_ref, w_ref, b_ref, p_ref, acc_prob, acc_z, j):
    r0 = j * _C
    l = lax.dot_general(
        x_ref[pl.ds(r0, _C), :], w_ref[...],
        dimension_numbers=(((1,), (1,)), ((), ())),
        preferred_element_type=jnp.float32,
    ) + b_ref[...]
    m = jnp.max(l, axis=-1, keepdims=True)
    e = jnp.exp(l - m)
    s = jnp.sum(e, axis=-1, keepdims=True)
    p = e * (1.0 / s)
    lse = m + jnp.log(s)
    p_ref[pl.ds(r0, _C), :] = p
    acc_prob[...] += jnp.sum(p, axis=0, keepdims=True)
    acc_z[...] += jnp.sum(lse * lse).reshape(1, 1)


def _tc_probs(x, W, b2, n_tok):
    n_blocks = n_tok // _T

    def body(x_ref, w_ref, b_ref, p_ref, sp_ref, z_ref, acc_prob, acc_z):
        i = pl.program_id(0)

        @pl.when(i == 0)
        def _init():
            acc_prob[...] = jnp.zeros_like(acc_prob)
            acc_z[...] = jnp.zeros_like(acc_z)

        for j in range(_T // _C):
            _tc_chunk(x_ref, w_ref, b_ref, p_ref, acc_prob, acc_z, j)

        @pl.when(i == n_blocks - 1)
        def _final():
            sp_ref[...] = acc_prob[...]
            z_ref[...] = acc_z[...]

    return pl.pallas_call(
        body,
        grid=(n_blocks,),
        in_specs=[
            pl.BlockSpec((_T, _D), lambda i: (i, 0)),
            pl.BlockSpec((_E, _D), lambda i: (0, 0)),
            pl.BlockSpec((1, _E), lambda i: (0, 0)),
        ],
        out_specs=[
            pl.BlockSpec((_T, _E), lambda i: (i, 0)),
            pl.BlockSpec((1, _E), lambda i: (0, 0)),
            pl.BlockSpec((1, 1), lambda i: (0, 0)),
        ],
        out_shape=[
            jax.ShapeDtypeStruct((n_tok, _E), jnp.float32),
            jax.ShapeDtypeStruct((1, _E), jnp.float32),
            jax.ShapeDtypeStruct((1, 1), jnp.float32),
        ],
        scratch_shapes=[
            pltpu.VMEM((1, _E), jnp.float32),
            pltpu.VMEM((1, 1), jnp.float32),
        ],
    )(x, W, b2)


def _sc_tail(p_flat):
    mesh = plsc.VectorSubcoreMesh(core_axis_name="c", subcore_axis_name="s")
    nelem = _RPW * _E
    n_groups = _RPW // 16

    def body(p_hbm, mask_hbm, wout_hbm, cnt_hbm, p_v, mask_v, wout_v, cnt_v):
        cid = lax.axis_index("c")
        sid = lax.axis_index("s")
        wid = sid * 2 + cid
        base = wid * nelem
        pltpu.sync_copy(p_hbm.at[pl.ds(base, nelem)], p_v)

        zero = jnp.zeros((16,), jnp.float32)

        def allmax(x):
            return jnp.maximum(
                plsc.cummax(x),
                lax.rev(plsc.cummax(lax.rev(x, (0,))), (0,)))

        def one_row(o, cnt):
            c0, c1, c2, c3 = cnt
            v0 = p_v[pl.ds(o, 16)]
            v1 = p_v[pl.ds(o + 16, 16)]
            v2 = p_v[pl.ds(o + 32, 16)]
            v3 = p_v[pl.ds(o + 48, 16)]
            t = jnp.maximum(jnp.maximum(v0, v1), jnp.maximum(v2, v3))
            m1 = allmax(t)
            w0 = jnp.where(v0 >= m1, -1.0, v0)
            w1 = jnp.where(v1 >= m1, -1.0, v1)
            w2 = jnp.where(v2 >= m1, -1.0, v2)
            w3 = jnp.where(v3 >= m1, -1.0, v3)
            u = jnp.maximum(jnp.maximum(w0, w1), jnp.maximum(w2, w3))
            m2 = allmax(u)
            k0 = jnp.where(v0 >= m2, 1.0, 0.0)
            k1 = jnp.where(v1 >= m2, 1.0, 0.0)
            k2 = jnp.where(v2 >= m2, 1.0, 0.0)
            k3 = jnp.where(v3 >= m2, 1.0, 0.0)
            mask_v[pl.ds(o, 16)] = k0
            mask_v[pl.ds(o + 16, 16)] = k1
            mask_v[pl.ds(o + 32, 16)] = k2
            mask_v[pl.ds(o + 48, 16)] = k3
            wout_v[pl.ds(o, 16)] = v0 * k0
            wout_v[pl.ds(o + 16, 16)] = v1 * k1
            wout_v[pl.ds(o + 32, 16)] = v2 * k2
            wout_v[pl.ds(o + 48, 16)] = v3 * k3
            return (c0 + k0, c1 + k1, c2 + k2, c3 + k3)

        def pair(r, cnt):
            o = r * (8 * _E)
            for q in range(8):
                cnt = one_row(o + q * _E, cnt)
            return cnt

        cnt = lax.fori_loop(0, _RPW // 8, pair, (zero, zero, zero, zero))
        c0, c1, c2, c3 = cnt
        cnt_v[pl.ds(0, 16)] = c0
        cnt_v[pl.ds(16, 16)] = c1
        cnt_v[pl.ds(32, 16)] = c2
        cnt_v[pl.ds(48, 16)] = c3
        pltpu.sync_copy(mask_v, mask_hbm.at[pl.ds(base, nelem)])
        pltpu.sync_copy(wout_v, wout_hbm.at[pl.ds(base, nelem)])
        pltpu.sync_copy(cnt_v, cnt_hbm.at[wid])

    return pl.kernel(
        body,
        out_type=[
            jax.ShapeDtypeStruct((_N * _E,), jnp.float32),
            jax.ShapeDtypeStruct((_N * _E,), jnp.float32),
            jax.ShapeDtypeStruct((_NW, _E), jnp.float32),
        ],
        mesh=mesh,
        compiler_params=pltpu.CompilerParams(needs_layout_passes=False),
        scratch_types=[
            pltpu.VMEM((nelem,), jnp.float32),
            pltpu.VMEM((nelem,), jnp.float32),
            pltpu.VMEM((nelem,), jnp.float32),
            pltpu.VMEM((_E,), jnp.float32),
        ],
    )(p_flat)


def _tc_loss(cnt, sp, z, n_tok):
    def body(cnt_ref, sp_ref, z_ref, loss_ref):
        inv_n = 1.0 / n_tok
        frac_tok = jnp.sum(cnt_ref[...], axis=0, keepdims=True) * inv_n
        aux = _ALPHA * _E * jnp.sum(frac_tok * sp_ref[...] * inv_n)
        loss_ref[...] = aux.reshape(1, 1) + z_ref[...] * inv_n

    return pl.pallas_call(
        body,
        out_shape=jax.ShapeDtypeStruct((1, 1), jnp.float32),
    )(cnt, sp, z)


def kernel(inputs, W, b):
    orig_dtype = inputs.dtype
    x = inputs.astype(jnp.float32).reshape(-1, _D)
    n_tok = x.shape[0]
    b2 = b.reshape(1, _E).astype(jnp.float32)

    p, sp, z = _tc_probs(x, W, b2, n_tok)
    mask_f, wout_f, cnt = _sc_tail(p.reshape(-1))
    loss = _tc_loss(cnt, sp, z, n_tok)

    mask = mask_f.reshape(n_tok, _E)
    wout = wout_f.reshape(n_tok, _E).astype(orig_dtype)
    return (mask, wout, loss[0, 0])


# speedup vs baseline: 1.8100x; 1.8100x over previous
import jax
import jax.numpy as jnp
from jax import lax
from jax.experimental import pallas as pl
from jax.experimental.pallas import tpu as pltpu

_E = 64
_D = 2048
_ALPHA = 0.01
_T = 2048
_C = 512


def _chunk(x_ref, w_ref, b_ref, mask_ref, wout_ref, acc_mask, acc_prob,
           acc_z, j):
    r0 = j * _C
    l = lax.dot_general(
        x_ref[pl.ds(r0, _C), :], w_ref[...],
        dimension_numbers=(((1,), (1,)), ((), ())),
        preferred_element_type=jnp.float32,
    ) + b_ref[...]

    m = jnp.max(l, axis=-1, keepdims=True)
    e = jnp.exp(l - m)
    s = jnp.sum(e, axis=-1, keepdims=True)
    p = e * (1.0 / s)
    lse = m + jnp.log(s)

    sel1 = e >= 1.0
    m2 = jnp.max(jnp.where(sel1, -1.0, e), axis=-1, keepdims=True)
    mask = jnp.where(sel1 | (e >= m2), 1.0, 0.0).astype(jnp.float32)

    mask_ref[pl.ds(r0, _C), :] = mask
    wout_ref[pl.ds(r0, _C), :] = p * mask

    acc_mask[...] += jnp.sum(mask, axis=0, keepdims=True)
    acc_prob[...] += jnp.sum(p, axis=0, keepdims=True)
    acc_z[...] += jnp.sum(lse * lse).reshape(1, 1)


def kernel(inputs, W, b):
    orig_dtype = inputs.dtype
    x = inputs.astype(jnp.float32).reshape(-1, _D)
    n_tok = x.shape[0]
    n_blocks = n_tok // _T
    b2 = b.reshape(1, _E).astype(jnp.float32)

    def body(x_ref, w_ref, b_ref, mask_ref, wout_ref, loss_ref,
             acc_mask, acc_prob, acc_z):
        i = pl.program_id(0)

        @pl.when(i == 0)
        def _init():
            acc_mask[...] = jnp.zeros_like(acc_mask)
            acc_prob[...] = jnp.zeros_like(acc_prob)
            acc_z[...] = jnp.zeros_like(acc_z)

        for j in range(_T // _C):
            _chunk(x_ref, w_ref, b_ref, mask_ref, wout_ref,
                   acc_mask, acc_prob, acc_z, j)

        @pl.when(i == n_blocks - 1)
        def _final():
            inv_n = 1.0 / n_tok
            aux = _ALPHA * _E * jnp.sum(
                (acc_mask[...] * inv_n) * (acc_prob[...] * inv_n))
            loss_ref[...] = aux.reshape(1, 1) + acc_z[...] * inv_n

    mask, wout, loss = pl.pallas_call(
        body,
        grid=(n_blocks,),
        in_specs=[
            pl.BlockSpec((_T, _D), lambda i: (i, 0)),
            pl.BlockSpec((_E, _D), lambda i: (0, 0)),
            pl.BlockSpec((1, _E), lambda i: (0, 0)),
        ],
        out_specs=[
            pl.BlockSpec((_T, _E), lambda i: (i, 0)),
            pl.BlockSpec((_T, _E), lambda i: (i, 0)),
            pl.BlockSpec((1, 1), lambda i: (0, 0)),
        ],
        out_shape=[
            jax.ShapeDtypeStruct((n_tok, _E), jnp.float32),
            jax.ShapeDtypeStruct((n_tok, _E), jnp.float32),
            jax.ShapeDtypeStruct((1, 1), jnp.float32),
        ],
        scratch_shapes=[
            pltpu.VMEM((1, _E), jnp.float32),
            pltpu.VMEM((1, _E), jnp.float32),
            pltpu.VMEM((1, 1), jnp.float32),
        ],
    )(x, W, b2)

    return (mask, wout.astype(orig_dtype), loss[0, 0])
